# baseline (speedup 1.0000x reference)
import functools

import jax
import jax.numpy as jnp
from jax.experimental import pallas as pl
from jax.experimental.pallas import tpu as pltpu


def _mp_body(adj_ref, ne_ref, er_ref, dexp_ref, cw_ref, w1t_ref, bmsg_ref,
             wuat_ref, wubt_ref, bupd_ref, out_ref):
    f32 = jnp.float32
    maskf = (adj_ref[...] > 0).astype(f32)
    ne = ne_ref[0]
    er = er_ref[0]
    z = jnp.dot(maskf, er, preferred_element_type=f32)
    term2 = jnp.dot(z * dexp_ref[...], cw_ref[...],
                    preferred_element_type=f32)
    pre = jnp.dot(ne, w1t_ref[...], preferred_element_type=f32) + bmsg_ref[...]
    msgs = jnp.dot(maskf, pre, preferred_element_type=f32) + term2
    h = (jnp.dot(ne, wuat_ref[...], preferred_element_type=f32)
         + jnp.dot(msgs, wubt_ref[...], preferred_element_type=f32)
         + bupd_ref[...])
    out_ref[0] = jnp.maximum(h, 0.0)


@functools.partial(jax.jit, static_argnames=())
def _run(node_embeddings, edge_relations, adjacency, W_msg, b_msg, W_upd,
         b_upd):
    B, N, H = node_embeddings.shape
    E = edge_relations.shape[-1]
    er2 = edge_relations.reshape(B, N, N * E)
    W1T = W_msg[:, :H].T
    W2T = W_msg[:, H:].T
    CW = jnp.tile(W2T, (N, 1))
    col_j = jax.lax.broadcasted_iota(jnp.int32, (N, N * E), 1) // E
    row_j = jax.lax.broadcasted_iota(jnp.int32, (N, N * E), 0)
    Dexp = (col_j == row_j).astype(jnp.float32)
    WuAT = W_upd[:, :H].T
    WuBT = W_upd[:, H:].T
    bmsg2 = b_msg.reshape(1, H)
    bupd2 = b_upd.reshape(1, H)

    grid = (B,)
    out = pl.pallas_call(
        _mp_body,
        grid=grid,
        in_specs=[
            pl.BlockSpec((N, N), lambda b: (0, 0)),
            pl.BlockSpec((1, N, H), lambda b: (b, 0, 0)),
            pl.BlockSpec((1, N, N * E), lambda b: (b, 0, 0)),
            pl.BlockSpec((N, N * E), lambda b: (0, 0)),
            pl.BlockSpec((N * E, H), lambda b: (0, 0)),
            pl.BlockSpec((H, H), lambda b: (0, 0)),
            pl.BlockSpec((1, H), lambda b: (0, 0)),
            pl.BlockSpec((H, H), lambda b: (0, 0)),
            pl.BlockSpec((H, H), lambda b: (0, 0)),
            pl.BlockSpec((1, H), lambda b: (0, 0)),
        ],
        out_specs=pl.BlockSpec((1, N, H), lambda b: (b, 0, 0)),
        out_shape=jax.ShapeDtypeStruct((B, N, H), jnp.float32),
        compiler_params=pltpu.CompilerParams(
            dimension_semantics=("arbitrary",)),
    )(adjacency, node_embeddings, er2, Dexp, CW, W1T, bmsg2, WuAT, WuBT,
      bupd2)
    return out


def kernel(node_embeddings, edge_relations, adjacency, W_msg, b_msg, W_upd,
           b_upd):
    return _run(node_embeddings, edge_relations, adjacency, W_msg, b_msg,
                W_upd, b_upd)
